# baseline (speedup 1.0000x reference)
import functools

import jax
import jax.numpy as jnp
from jax import lax
from jax.experimental import pallas as pl
from jax.experimental.pallas import tpu as pltpu
from jax.experimental.pallas import tpu_sc as plsc

N_NODES = 10000
N_EDGES = 160000
LATENT = 128
N_BLOCKS = 15

NC = 2
NS = 16
NW = NC * NS
CH = 128
EPAD = 163840
EH = EPAD // 2
GCH = 2 * EH // (NW * CH)
GPW = GCH * CH
SCH = EH // (NW * CH)
GNB = 5
GNB2 = 5
SNB = 2

ACC_ROWS = 10240
TRASH_ROW = 10000

_BF = jnp.bfloat16


@functools.cache
def _sc_mesh():
    return plsc.VectorSubcoreMesh(core_axis_name="c", subcore_axis_name="s",
                                  num_cores=NC, num_subcores=NS)


@functools.partial(jax.jit, static_argnames=("dt", "d"))
def _sc_gather2(tab2, idx2, dt, d):
    cp = None
    nb = GNB
    if d != LATENT:
        cp = pltpu.CompilerParams(use_tc_tiling_on_sc=False)
        nb = GNB2

    @functools.partial(
        pl.kernel,
        out_type=jax.ShapeDtypeStruct((2, EH, d), dt),
        mesh=_sc_mesh(),
        compiler_params=cp,
        scratch_types=(
            [pltpu.VMEM((GCH, CH), jnp.int32)]
            + [pltpu.VMEM((CH, d), dt)] * nb
            + [pltpu.SemaphoreType.DMA] * (2 * nb)
        ),
    )
    def k(tab_h, idx_h, out_h, idx_v, *scr):
        GNB = nb
        bufs = scr[:GNB]
        gsem = scr[GNB:2 * GNB]
        ssem = scr[2 * GNB:]
        wid = lax.axis_index("s") * NC + lax.axis_index("c")
        q = wid // NS
        r0 = (wid % NS) * GPW
        pltpu.sync_copy(idx_h.at[wid], idx_v)

        def start_gather(b, j):
            pltpu.async_copy(tab_h.at[idx_v.at[j]], bufs[b], gsem[b])

        def wait_gather(b, j):
            pltpu.make_async_copy(tab_h.at[idx_v.at[j]], bufs[b], gsem[b]).wait()

        def start_store(b, j):
            pltpu.async_copy(bufs[b], out_h.at[q, pl.ds(r0 + j * CH, CH)], ssem[b])

        def wait_store(b, j):
            pltpu.make_async_copy(bufs[b], out_h.at[q, pl.ds(r0 + j * CH, CH)],
                                  ssem[b]).wait()

        for b in range(GNB):
            start_gather(b, b)

        ngroup = GCH // GNB

        def group(i, carry):
            for b in range(GNB):
                j = i * GNB + b
                wait_gather(b, j)
                start_store(b, j)
            for b in range(GNB):
                j = i * GNB + b
                wait_store(b, j)
                start_gather(b, j + GNB)
            return carry

        lax.fori_loop(0, ngroup - 1, group, 0)
        for b in range(GNB):
            j = (ngroup - 1) * GNB + b
            wait_gather(b, j)
            start_store(b, j)
        for b in range(GNB):
            j = (ngroup - 1) * GNB + b
            wait_store(b, j)

    return k(tab2, idx2)


@jax.jit
def _sc_scatter_add(vals, idx, zeros128):
    @functools.partial(
        pl.kernel,
        out_type=jax.ShapeDtypeStruct((NC, ACC_ROWS, LATENT), jnp.float32),
        mesh=_sc_mesh(),
        scratch_types=(
            [pltpu.VMEM((SCH, CH), jnp.int32)]
            + [pltpu.VMEM((CH, LATENT), jnp.float32)] * SNB
            + [pltpu.VMEM_SHARED((ACC_ROWS, LATENT), jnp.float32),
               pltpu.SemaphoreType.DMA]
        ),
    )
    def k(vals_h, idx_h, zeros_h, out_h, idx_v, *scr):
        bufs = scr[:SNB]
        acc_s = scr[SNB]
        sem = scr[SNB + 1]
        c = lax.axis_index("c")
        s = lax.axis_index("s")
        pltpu.sync_copy(zeros_h, bufs[0])
        stripe = ACC_ROWS // NS
        for t in range(stripe // CH):
            pltpu.sync_copy(bufs[0], acc_s.at[pl.ds(s * stripe + t * CH, CH)])
        plsc.subcore_barrier()

        pltpu.sync_copy(idx_h.at[c, s], idx_v)
        base = (c * NS + s) * (SCH * CH)

        def start_load(b, j):
            pltpu.async_copy(vals_h.at[pl.ds(base + j * CH, CH)], bufs[b], sem)

        def wait_load(b, j):
            pltpu.make_async_copy(vals_h.at[pl.ds(base + j * CH, CH)], bufs[b],
                                  sem).wait()

        for b in range(SNB):
            start_load(b, b)

        def group(i, carry):
            for b in range(SNB):
                j = i * SNB + b
                wait_load(b, j)
                pltpu.sync_copy(bufs[b], acc_s.at[idx_v.at[j]], add=True)
                start_load(b, j + SNB)
            return carry

        lax.fori_loop(0, SCH // SNB - 1, group, 0)
        for b in range(SNB):
            j = SCH - SNB + b
            wait_load(b, j)
            pltpu.sync_copy(bufs[b], acc_s.at[idx_v.at[j]], add=True)
        plsc.subcore_barrier()

        for t in range(stripe // CH):
            r0 = s * stripe + t * CH
            pltpu.sync_copy(acc_s.at[pl.ds(r0, CH)], bufs[0])
            pltpu.sync_copy(bufs[0], out_h.at[c, pl.ds(r0, CH)])

    return k(vals, idx, zeros128)


def _pack_bf16(lo_f32, hi_f32):
    lo = lax.bitcast_convert_type(lo_f32.astype(_BF).astype(jnp.float32), jnp.int32)
    hi = lax.bitcast_convert_type(hi_f32.astype(_BF).astype(jnp.float32), jnp.int32)
    return jnp.bitwise_or(lax.shift_right_logical(lo, 16),
                          jnp.bitwise_and(hi, jnp.int32(-65536)))


def _unpack_bf16(p):
    lo = lax.bitcast_convert_type(lax.shift_left(p, 16), jnp.float32)
    hi = lax.bitcast_convert_type(jnp.bitwise_and(p, jnp.int32(-65536)), jnp.float32)
    return lo, hi


def _ln(h, g, b):
    m = jnp.mean(h, axis=-1, keepdims=True)
    v = jnp.mean((h - m) * (h - m), axis=-1, keepdims=True)
    return (h - m) * lax.rsqrt(v + 1e-5) * g + b


def _full_spec(*shape):
    return pl.BlockSpec(shape, lambda i: tuple(0 for _ in shape))


_BN = 2000
_BE = 2048


@jax.jit
def _tc_node_encoder(feat16, mean16, sinv16, w1, w2, w3, vecs, wa, wb):
    def body(f, m, si, w1r, w2r, w3r, vr, war, wbr, x_o, xab_o):
        f_n = (f[...] - m[...]) * si[...]
        h = jnp.maximum(jnp.dot(f_n, w1r[...], preferred_element_type=jnp.float32) + vr[0:1], 0.0)
        h = jnp.maximum(jnp.dot(h, w2r[...], preferred_element_type=jnp.float32) + vr[1:2], 0.0)
        h = jnp.dot(h, w3r[...], preferred_element_type=jnp.float32) + vr[2:3]
        x = _ln(h, vr[3:4], vr[4:5])
        x_o[...] = x
        xa = jnp.dot(x, war[...], preferred_element_type=jnp.float32)
        xb = jnp.dot(x, wbr[...], preferred_element_type=jnp.float32)
        xab_o[0] = _pack_bf16(xa[:, :64], xa[:, 64:])
        xab_o[1] = _pack_bf16(xb[:, :64], xb[:, 64:])

    grid = (N_NODES // _BN,)
    return pl.pallas_call(
        body,
        grid=grid,
        in_specs=[
            pl.BlockSpec((_BN, 16), lambda i: (i, 0)),
            _full_spec(1, 16), _full_spec(1, 16),
            _full_spec(16, LATENT), _full_spec(LATENT, LATENT), _full_spec(LATENT, LATENT),
            _full_spec(8, LATENT),
            _full_spec(LATENT, LATENT), _full_spec(LATENT, LATENT),
        ],
        out_specs=[pl.BlockSpec((_BN, LATENT), lambda i: (i, 0)),
                   pl.BlockSpec((NC, _BN, LATENT // 2), lambda i: (0, i, 0))],
        out_shape=[jax.ShapeDtypeStruct((N_NODES, LATENT), jnp.float32),
                   jax.ShapeDtypeStruct((NC, N_NODES, LATENT // 2), jnp.int32)],
        compiler_params=pltpu.CompilerParams(dimension_semantics=("parallel",)),
    )(feat16, mean16, sinv16, w1, w2, w3, vecs, wa, wb)


@jax.jit
def _tc_edge_encoder(psd, mean128, sinv128, w1, w2, w3, vecs):
    def body(ps_r, pd_r, m, si, w1r, w2r, w3r, vr, e_o):
        rel = ps_r[0] - pd_r[0]
        lane = lax.broadcasted_iota(jnp.int32, (_BE, LATENT), 1)
        sq = rel * rel
        s_w = jnp.sum(jnp.where(lane < 4, sq, 0.0), axis=-1, keepdims=True)
        s_all = jnp.sum(sq, axis=-1, keepdims=True)
        nw = jnp.sqrt(s_w)
        nm = jnp.sqrt(jnp.maximum(s_all - s_w, 0.0))
        ef = rel + jnp.where(lane == 3, nw, 0.0) + jnp.where(lane == 7, nm, 0.0)
        f_n = (ef - m[...]) * si[...]
        h = jnp.maximum(jnp.dot(f_n, w1r[...], preferred_element_type=jnp.float32) + vr[0:1], 0.0)
        h = jnp.maximum(jnp.dot(h, w2r[...], preferred_element_type=jnp.float32) + vr[1:2], 0.0)
        h = jnp.dot(h, w3r[...], preferred_element_type=jnp.float32) + vr[2:3]
        e_o[...] = _ln(h, vr[3:4], vr[4:5])

    grid = (EH // _BE,)
    return pl.pallas_call(
        body,
        grid=grid,
        in_specs=[
            pl.BlockSpec((1, _BE, LATENT), lambda i: (0, i, 0)),
            pl.BlockSpec((1, _BE, LATENT), lambda i: (1, i, 0)),
            _full_spec(1, LATENT), _full_spec(1, LATENT),
            _full_spec(LATENT, LATENT), _full_spec(LATENT, LATENT), _full_spec(LATENT, LATENT),
            _full_spec(8, LATENT),
        ],
        out_specs=pl.BlockSpec((_BE, LATENT), lambda i: (i, 0)),
        out_shape=jax.ShapeDtypeStruct((EH, LATENT), jnp.float32),
        compiler_params=pltpu.CompilerParams(dimension_semantics=("parallel",)),
    )(psd, psd, mean128, sinv128, w1, w2, w3, vecs)


@jax.jit
def _tc_edge_block(xsd, e, w1c, w2, w3, vecs):
    def body(xs_r, xd_r, e_r, w1r, w2r, w3r, vr, o):
        e_in = e_r[...]
        s_lo, s_hi = _unpack_bf16(xs_r[0])
        d_lo, d_hi = _unpack_bf16(xd_r[0])
        g = jnp.concatenate([s_lo + d_lo, s_hi + d_hi], axis=-1)
        h = g + jnp.dot(e_in.astype(_BF), w1r[...], preferred_element_type=jnp.float32)
        h = jnp.maximum(h + vr[0:1], 0.0)
        h = jnp.maximum(jnp.dot(h.astype(_BF), w2r[...], preferred_element_type=jnp.float32) + vr[1:2], 0.0)
        h = jnp.dot(h.astype(_BF), w3r[...], preferred_element_type=jnp.float32) + vr[2:3]
        o[...] = _ln(h, vr[3:4], vr[4:5]) + e_in

    grid = (EH // _BE,)
    espec = pl.BlockSpec((_BE, LATENT), lambda i: (i, 0))
    return pl.pallas_call(
        body,
        grid=grid,
        in_specs=[pl.BlockSpec((1, _BE, LATENT // 2), lambda i: (0, i, 0)),
                  pl.BlockSpec((1, _BE, LATENT // 2), lambda i: (1, i, 0)),
                  espec,
                  _full_spec(LATENT, LATENT), _full_spec(LATENT, LATENT),
                  _full_spec(LATENT, LATENT), _full_spec(8, LATENT)],
        out_specs=espec,
        out_shape=jax.ShapeDtypeStruct((EH, LATENT), jnp.float32),
        compiler_params=pltpu.CompilerParams(dimension_semantics=("parallel",)),
    )(xsd, xsd, e, w1c.astype(_BF), w2.astype(_BF), w3.astype(_BF), vecs)


@jax.jit
def _tc_node_block(x, agg_a, agg_b, wn1a, wn1b, wn2, wn3, vecs, wa, wb):
    def body(x_r, aa_r, ab_r, w1a, w1b, w2r, w3r, vr, war, wbr, x_o, xab_o):
        x_in = x_r[...]
        agg = aa_r[0] + aa_r[1] + ab_r[0] + ab_r[1]
        h = (jnp.dot(x_in, w1a[...], preferred_element_type=jnp.float32)
             + jnp.dot(agg, w1b[...], preferred_element_type=jnp.float32))
        h = jnp.maximum(h + vr[0:1], 0.0)
        h = jnp.maximum(jnp.dot(h, w2r[...], preferred_element_type=jnp.float32) + vr[1:2], 0.0)
        h = jnp.dot(h, w3r[...], preferred_element_type=jnp.float32) + vr[2:3]
        x_new = _ln(h, vr[3:4], vr[4:5]) + x_in
        x_o[...] = x_new
        xa = jnp.dot(x_new, war[...], preferred_element_type=jnp.float32)
        xb = jnp.dot(x_new, wbr[...], preferred_element_type=jnp.float32)
        xab_o[0] = _pack_bf16(xa[:, :64], xa[:, 64:])
        xab_o[1] = _pack_bf16(xb[:, :64], xb[:, 64:])

    grid = (N_NODES // _BN,)
    nspec = pl.BlockSpec((_BN, LATENT), lambda i: (i, 0))
    aspec = pl.BlockSpec((NC, _BN, LATENT), lambda i: (0, i, 0))
    return pl.pallas_call(
        body,
        grid=grid,
        in_specs=[nspec, aspec, aspec,
                  _full_spec(LATENT, LATENT), _full_spec(LATENT, LATENT),
                  _full_spec(LATENT, LATENT), _full_spec(LATENT, LATENT),
                  _full_spec(8, LATENT),
                  _full_spec(LATENT, LATENT), _full_spec(LATENT, LATENT)],
        out_specs=[nspec, pl.BlockSpec((NC, _BN, LATENT // 2), lambda i: (0, i, 0))],
        out_shape=[jax.ShapeDtypeStruct((N_NODES, LATENT), jnp.float32),
                   jax.ShapeDtypeStruct((NC, N_NODES, LATENT // 2), jnp.int32)],
        compiler_params=pltpu.CompilerParams(dimension_semantics=("parallel",)),
    )(x, agg_a, agg_b, wn1a, wn1b, wn2, wn3, vecs, wa, wb)


@jax.jit
def _tc_decoder(x, w1, w2, w3p, vecs):
    def body(x_r, w1r, w2r, w3r, vr, o):
        h = jnp.maximum(jnp.dot(x_r[...], w1r[...], preferred_element_type=jnp.float32) + vr[0:1], 0.0)
        h = jnp.maximum(jnp.dot(h, w2r[...], preferred_element_type=jnp.float32) + vr[1:2], 0.0)
        o[...] = jnp.dot(h, w3r[...], preferred_element_type=jnp.float32) + vr[2:3]

    grid = (N_NODES // _BN,)
    nspec = pl.BlockSpec((_BN, LATENT), lambda i: (i, 0))
    return pl.pallas_call(
        body,
        grid=grid,
        in_specs=[nspec,
                  _full_spec(LATENT, LATENT), _full_spec(LATENT, LATENT),
                  _full_spec(LATENT, LATENT), _full_spec(8, LATENT)],
        out_specs=nspec,
        out_shape=jax.ShapeDtypeStruct((N_NODES, LATENT), jnp.float32),
        compiler_params=pltpu.CompilerParams(dimension_semantics=("parallel",)),
    )(x, w1, w2, w3p, vecs)


def _stats_mean_std(stats):
    c = jnp.maximum(stats['count'], 1.0)
    mean = stats['sum'] / c
    std = jnp.sqrt(jnp.maximum(stats['sumsq'] / c - mean ** 2, 1e-8))
    return mean, jnp.maximum(std, 1e-8)


def _pad16(v, n):
    out = jnp.zeros((16,), jnp.float32)
    return out.at[:n].set(v).reshape(1, 16)


def _vecs(b1, b2, b3, g, bb):
    z = jnp.zeros((LATENT,), jnp.float32)
    return jnp.stack([b1, b2, b3, g, bb, z, z, z])


def kernel(world_pos, prev_world_pos, mesh_pos, params, node_type, edge_index):
    src = edge_index[0].astype(jnp.int32)
    dst = edge_index[1].astype(jnp.int32)
    npad = EPAD - N_EDGES
    src_p = jnp.concatenate([src, jnp.zeros((npad,), jnp.int32)])
    dst_pg = jnp.concatenate([dst + N_NODES,
                              jnp.full((npad,), N_NODES, jnp.int32)])
    dst_ps = jnp.concatenate([dst, jnp.full((npad,), TRASH_ROW, jnp.int32)])
    idx_g = []
    dst_s = []
    for h in range(2):
        sl = slice(h * EH, (h + 1) * EH)
        idx_g.append(jnp.concatenate([src_p[sl], dst_pg[sl]])
                     .reshape(NW, GCH, CH))
        dst_s.append(dst_ps[sl].reshape(NC, NS, SCH, CH))
    zeros128 = jnp.zeros((CH, LATENT), jnp.float32)

    ptab = jnp.zeros((N_NODES, LATENT), jnp.float32)
    ptab = ptab.at[:, 0:3].set(world_pos).at[:, 4:7].set(mesh_pos)
    pt2 = jnp.concatenate([ptab, ptab], axis=0)

    vel = world_pos - prev_world_pos
    onehot = jax.nn.one_hot(node_type, 9, dtype=jnp.float32)
    feat16 = jnp.zeros((N_NODES, 16), jnp.float32)
    feat16 = feat16.at[:, 0:3].set(vel).at[:, 3:12].set(onehot)
    nmean, nstd = _stats_mean_std(params['node_norm'])
    nmean16 = _pad16(nmean, 12)
    nsinv16 = _pad16(1.0 / nstd, 12)

    emean, estd = _stats_mean_std(params['mesh_norm'])
    emean128 = jnp.zeros((1, LATENT), jnp.float32).at[0, :8].set(emean)
    esinv128 = jnp.zeros((1, LATENT), jnp.float32).at[0, :8].set(1.0 / estd)

    def mlp_w(ps):
        return [p[0] for p in ps], [p[1] for p in ps]

    w1_0 = params['blocks'][0]['edge_mlp'][0][0]
    enc_w, enc_b = mlp_w(params['enc_node'])
    w1n16 = jnp.zeros((16, LATENT), jnp.float32).at[:12].set(enc_w[0])
    x, xab = _tc_node_encoder(
        feat16, nmean16, nsinv16, w1n16, enc_w[1], enc_w[2],
        _vecs(enc_b[0], enc_b[1], enc_b[2], *params['enc_node_ln']),
        w1_0[:LATENT], w1_0[LATENT:2 * LATENT])

    ence_w, ence_b = mlp_w(params['enc_edge'])
    w1e128 = jnp.zeros((LATENT, LATENT), jnp.float32).at[:8].set(ence_w[0])
    evecs = _vecs(ence_b[0], ence_b[1], ence_b[2], *params['enc_edge_ln'])
    e = []
    for h in range(2):
        psd = _sc_gather2(pt2, idx_g[h], jnp.float32, LATENT)
        e.append(_tc_edge_encoder(psd, emean128, esinv128, w1e128,
                                  ence_w[1], ence_w[2], evecs))

    for b in range(N_BLOCKS):
        blk = params['blocks'][b]
        ew, ebias = mlp_w(blk['edge_mlp'])
        nw_, nbias = mlp_w(blk['node_mlp'])
        ev = _vecs(ebias[0], ebias[1], ebias[2], *blk['edge_ln'])
        tab = xab.reshape(2 * N_NODES, LATENT // 2)
        xsd0 = _sc_gather2(tab, idx_g[0], jnp.int32, LATENT // 2)
        xsd1 = _sc_gather2(tab, idx_g[1], jnp.int32, LATENT // 2)
        e[0] = _tc_edge_block(xsd0, e[0], ew[0][2 * LATENT:], ew[1], ew[2], ev)
        agg0 = _sc_scatter_add(e[0], dst_s[0], zeros128)
        e[1] = _tc_edge_block(xsd1, e[1], ew[0][2 * LATENT:], ew[1], ew[2], ev)
        agg1 = _sc_scatter_add(e[1], dst_s[1], zeros128)
        nxt = params['blocks'][(b + 1) % N_BLOCKS]['edge_mlp'][0][0]
        x, xab = _tc_node_block(
            x, agg0, agg1,
            nw_[0][:LATENT], nw_[0][LATENT:], nw_[1], nw_[2],
            _vecs(nbias[0], nbias[1], nbias[2], *blk['node_ln']),
            nxt[:LATENT], nxt[LATENT:2 * LATENT])

    omean, ostd = _stats_mean_std(params['out_norm'])
    dec_w, dec_b = mlp_w(params['dec'])
    w3p = jnp.zeros((LATENT, LATENT), jnp.float32).at[:, :3].set(dec_w[2] * ostd)
    b3p = jnp.zeros((LATENT,), jnp.float32).at[:3].set(dec_b[2] * ostd + omean)
    zl = jnp.zeros((LATENT,), jnp.float32)
    pred_pad = _tc_decoder(x, dec_w[0], dec_w[1], w3p,
                           _vecs(dec_b[0], dec_b[1], b3p, zl, zl))
    acc = pred_pad[:, :3]

    pred_pos = 2.0 * world_pos + acc - prev_world_pos
    mask = (node_type == 0)[:, None]
    new_prev = jnp.where(mask, world_pos, prev_world_pos)
    new_world = jnp.where(mask, pred_pos, world_pos)
    return new_world, new_prev


# speedup vs baseline: 1.0288x; 1.0288x over previous
import functools

import jax
import jax.numpy as jnp
from jax import lax
from jax.experimental import pallas as pl
from jax.experimental.pallas import tpu as pltpu
from jax.experimental.pallas import tpu_sc as plsc

N_NODES = 10000
N_EDGES = 160000
LATENT = 128
N_BLOCKS = 15

NC = 2
NS = 16
NW = NC * NS
CH = 128
EPAD = 163840
EH = EPAD // 2
GCH = 2 * EH // (NW * CH)
GPW = GCH * CH
SCH = EH // (NW * CH)
GNB = 5
GNB2 = 5
SNB = 2

ACC_ROWS = 10240
TRASH_ROW = 10000

_BF = jnp.bfloat16


@functools.cache
def _sc_mesh():
    return plsc.VectorSubcoreMesh(core_axis_name="c", subcore_axis_name="s",
                                  num_cores=NC, num_subcores=NS)


@functools.partial(jax.jit, static_argnames=("dt", "d"))
def _sc_gather2(tab2, idx2, dt, d):
    cp = None
    nb = GNB
    if d != LATENT:
        cp = pltpu.CompilerParams(use_tc_tiling_on_sc=False)
        nb = GNB2

    @functools.partial(
        pl.kernel,
        out_type=jax.ShapeDtypeStruct((2, EH, d), dt),
        mesh=_sc_mesh(),
        compiler_params=cp,
        scratch_types=(
            [pltpu.VMEM((GCH, CH), jnp.int32)]
            + [pltpu.VMEM((CH, d), dt)] * nb
            + [pltpu.SemaphoreType.DMA] * (2 * nb)
        ),
    )
    def k(tab_h, idx_h, out_h, idx_v, *scr):
        GNB = nb
        bufs = scr[:GNB]
        gsem = scr[GNB:2 * GNB]
        ssem = scr[2 * GNB:]
        wid = lax.axis_index("s") * NC + lax.axis_index("c")
        q = wid // NS
        r0 = (wid % NS) * GPW
        pltpu.sync_copy(idx_h.at[wid], idx_v)

        def start_gather(b, j):
            pltpu.async_copy(tab_h.at[idx_v.at[j]], bufs[b], gsem[b])

        def wait_gather(b, j):
            pltpu.make_async_copy(tab_h.at[idx_v.at[j]], bufs[b], gsem[b]).wait()

        def start_store(b, j):
            pltpu.async_copy(bufs[b], out_h.at[q, pl.ds(r0 + j * CH, CH)], ssem[b])

        def wait_store(b, j):
            pltpu.make_async_copy(bufs[b], out_h.at[q, pl.ds(r0 + j * CH, CH)],
                                  ssem[b]).wait()

        for b in range(GNB):
            start_gather(b, b)

        ngroup = GCH // GNB

        def group(i, carry):
            for b in range(GNB):
                j = i * GNB + b
                wait_gather(b, j)
                start_store(b, j)
            for b in range(GNB):
                j = i * GNB + b
                wait_store(b, j)
                start_gather(b, j + GNB)
            return carry

        lax.fori_loop(0, ngroup - 1, group, 0)
        for b in range(GNB):
            j = (ngroup - 1) * GNB + b
            wait_gather(b, j)
            start_store(b, j)
        for b in range(GNB):
            j = (ngroup - 1) * GNB + b
            wait_store(b, j)

    return k(tab2, idx2)


@jax.jit
def _sc_scatter_add(vals, idx, zeros128):
    @functools.partial(
        pl.kernel,
        out_type=jax.ShapeDtypeStruct((NC, ACC_ROWS, LATENT), jnp.float32),
        mesh=_sc_mesh(),
        scratch_types=(
            [pltpu.VMEM((SCH, CH), jnp.int32)]
            + [pltpu.VMEM((CH, LATENT), jnp.float32)] * SNB
            + [pltpu.VMEM_SHARED((ACC_ROWS, LATENT), jnp.float32),
               pltpu.SemaphoreType.DMA]
        ),
    )
    def k(vals_h, idx_h, zeros_h, out_h, idx_v, *scr):
        bufs = scr[:SNB]
        acc_s = scr[SNB]
        sem = scr[SNB + 1]
        c = lax.axis_index("c")
        s = lax.axis_index("s")
        pltpu.sync_copy(zeros_h, bufs[0])
        stripe = ACC_ROWS // NS
        for t in range(stripe // CH):
            pltpu.sync_copy(bufs[0], acc_s.at[pl.ds(s * stripe + t * CH, CH)])
        plsc.subcore_barrier()

        pltpu.sync_copy(idx_h.at[c, s], idx_v)
        base = (c * NS + s) * (SCH * CH)

        def start_load(b, j):
            pltpu.async_copy(vals_h.at[pl.ds(base + j * CH, CH)], bufs[b], sem)

        def wait_load(b, j):
            pltpu.make_async_copy(vals_h.at[pl.ds(base + j * CH, CH)], bufs[b],
                                  sem).wait()

        for b in range(SNB):
            start_load(b, b)

        def group(i, carry):
            for b in range(SNB):
                j = i * SNB + b
                wait_load(b, j)
                pltpu.sync_copy(bufs[b], acc_s.at[idx_v.at[j]], add=True)
                start_load(b, j + SNB)
            return carry

        lax.fori_loop(0, SCH // SNB - 1, group, 0)
        for b in range(SNB):
            j = SCH - SNB + b
            wait_load(b, j)
            pltpu.sync_copy(bufs[b], acc_s.at[idx_v.at[j]], add=True)
        plsc.subcore_barrier()

        for t in range(stripe // CH):
            r0 = s * stripe + t * CH
            pltpu.sync_copy(acc_s.at[pl.ds(r0, CH)], bufs[0])
            pltpu.sync_copy(bufs[0], out_h.at[c, pl.ds(r0, CH)])

    return k(vals, idx, zeros128)


def _pack_bf16(lo_f32, hi_f32):
    lo = lax.bitcast_convert_type(lo_f32.astype(_BF).astype(jnp.float32), jnp.int32)
    hi = lax.bitcast_convert_type(hi_f32.astype(_BF).astype(jnp.float32), jnp.int32)
    return jnp.bitwise_or(lax.shift_right_logical(lo, 16),
                          jnp.bitwise_and(hi, jnp.int32(-65536)))


def _unpack_bf16(p):
    lo = lax.bitcast_convert_type(lax.shift_left(p, 16), jnp.float32)
    hi = lax.bitcast_convert_type(jnp.bitwise_and(p, jnp.int32(-65536)), jnp.float32)
    return lo, hi


def _ln(h, g, b):
    m = jnp.mean(h, axis=-1, keepdims=True)
    v = jnp.mean((h - m) * (h - m), axis=-1, keepdims=True)
    return (h - m) * lax.rsqrt(v + 1e-5) * g + b


def _full_spec(*shape):
    return pl.BlockSpec(shape, lambda i: tuple(0 for _ in shape))


_BN = 2000
_BE = 2048


@jax.jit
def _tc_node_encoder(feat16, mean16, sinv16, w1, w2, w3, vecs, wa, wb):
    def body(f, m, si, w1r, w2r, w3r, vr, war, wbr, x_o, xab_o):
        f_n = (f[...] - m[...]) * si[...]
        h = jnp.maximum(jnp.dot(f_n, w1r[...], preferred_element_type=jnp.float32) + vr[0:1], 0.0)
        h = jnp.maximum(jnp.dot(h, w2r[...], preferred_element_type=jnp.float32) + vr[1:2], 0.0)
        h = jnp.dot(h, w3r[...], preferred_element_type=jnp.float32) + vr[2:3]
        x = _ln(h, vr[3:4], vr[4:5])
        x_o[...] = x
        xa = jnp.dot(x, war[...], preferred_element_type=jnp.float32)
        xb = jnp.dot(x, wbr[...], preferred_element_type=jnp.float32)
        xab_o[0] = _pack_bf16(xa[:, :64], xa[:, 64:])
        xab_o[1] = _pack_bf16(xb[:, :64], xb[:, 64:])

    grid = (N_NODES // _BN,)
    return pl.pallas_call(
        body,
        grid=grid,
        in_specs=[
            pl.BlockSpec((_BN, 16), lambda i: (i, 0)),
            _full_spec(1, 16), _full_spec(1, 16),
            _full_spec(16, LATENT), _full_spec(LATENT, LATENT), _full_spec(LATENT, LATENT),
            _full_spec(8, LATENT),
            _full_spec(LATENT, LATENT), _full_spec(LATENT, LATENT),
        ],
        out_specs=[pl.BlockSpec((_BN, LATENT), lambda i: (i, 0)),
                   pl.BlockSpec((NC, _BN, LATENT // 2), lambda i: (0, i, 0))],
        out_shape=[jax.ShapeDtypeStruct((N_NODES, LATENT), jnp.float32),
                   jax.ShapeDtypeStruct((NC, N_NODES, LATENT // 2), jnp.int32)],
        compiler_params=pltpu.CompilerParams(dimension_semantics=("parallel",)),
    )(feat16, mean16, sinv16, w1, w2, w3, vecs, wa, wb)


@jax.jit
def _tc_edge_encoder(psd, mean128, sinv128, w1, w2, w3, vecs):
    def body(ps_r, pd_r, m, si, w1r, w2r, w3r, vr, e_o):
        rel = ps_r[0] - pd_r[0]
        lane = lax.broadcasted_iota(jnp.int32, (_BE, LATENT), 1)
        sq = rel * rel
        s_w = jnp.sum(jnp.where(lane < 4, sq, 0.0), axis=-1, keepdims=True)
        s_all = jnp.sum(sq, axis=-1, keepdims=True)
        nw = jnp.sqrt(s_w)
        nm = jnp.sqrt(jnp.maximum(s_all - s_w, 0.0))
        ef = rel + jnp.where(lane == 3, nw, 0.0) + jnp.where(lane == 7, nm, 0.0)
        f_n = (ef - m[...]) * si[...]
        h = jnp.maximum(jnp.dot(f_n, w1r[...], preferred_element_type=jnp.float32) + vr[0:1], 0.0)
        h = jnp.maximum(jnp.dot(h, w2r[...], preferred_element_type=jnp.float32) + vr[1:2], 0.0)
        h = jnp.dot(h, w3r[...], preferred_element_type=jnp.float32) + vr[2:3]
        e_o[...] = _ln(h, vr[3:4], vr[4:5])

    grid = (EH // _BE,)
    return pl.pallas_call(
        body,
        grid=grid,
        in_specs=[
            pl.BlockSpec((1, _BE, LATENT), lambda i: (0, i, 0)),
            pl.BlockSpec((1, _BE, LATENT), lambda i: (1, i, 0)),
            _full_spec(1, LATENT), _full_spec(1, LATENT),
            _full_spec(LATENT, LATENT), _full_spec(LATENT, LATENT), _full_spec(LATENT, LATENT),
            _full_spec(8, LATENT),
        ],
        out_specs=pl.BlockSpec((_BE, LATENT), lambda i: (i, 0)),
        out_shape=jax.ShapeDtypeStruct((EH, LATENT), jnp.float32),
        compiler_params=pltpu.CompilerParams(dimension_semantics=("parallel",)),
    )(psd, psd, mean128, sinv128, w1, w2, w3, vecs)


@jax.jit
def _tc_edge_block(xsd, e, w1c, w2, w3, vecs):
    def body(xs_r, xd_r, e_r, w1r, w2r, w3r, vr, o):
        e_in = e_r[...]
        s_lo, s_hi = _unpack_bf16(xs_r[0])
        d_lo, d_hi = _unpack_bf16(xd_r[0])
        g = jnp.concatenate([s_lo + d_lo, s_hi + d_hi], axis=-1)
        h = g + jnp.dot(e_in, w1r[...], preferred_element_type=jnp.float32)
        h = jnp.maximum(h + vr[0:1], 0.0)
        h = jnp.maximum(jnp.dot(h, w2r[...], preferred_element_type=jnp.float32) + vr[1:2], 0.0)
        h = jnp.dot(h, w3r[...], preferred_element_type=jnp.float32) + vr[2:3]
        o[...] = _ln(h, vr[3:4], vr[4:5]) + e_in

    grid = (EH // _BE,)
    espec = pl.BlockSpec((_BE, LATENT), lambda i: (i, 0))
    return pl.pallas_call(
        body,
        grid=grid,
        in_specs=[pl.BlockSpec((1, _BE, LATENT // 2), lambda i: (0, i, 0)),
                  pl.BlockSpec((1, _BE, LATENT // 2), lambda i: (1, i, 0)),
                  espec,
                  _full_spec(LATENT, LATENT), _full_spec(LATENT, LATENT),
                  _full_spec(LATENT, LATENT), _full_spec(8, LATENT)],
        out_specs=espec,
        out_shape=jax.ShapeDtypeStruct((EH, LATENT), jnp.float32),
        compiler_params=pltpu.CompilerParams(dimension_semantics=("parallel",)),
    )(xsd, xsd, e, w1c, w2, w3, vecs)


@jax.jit
def _tc_node_block(x, agg_a, agg_b, wn1a, wn1b, wn2, wn3, vecs, wa, wb):
    def body(x_r, aa_r, ab_r, w1a, w1b, w2r, w3r, vr, war, wbr, x_o, xab_o):
        x_in = x_r[...]
        agg = aa_r[0] + aa_r[1] + ab_r[0] + ab_r[1]
        h = (jnp.dot(x_in, w1a[...], preferred_element_type=jnp.float32)
             + jnp.dot(agg, w1b[...], preferred_element_type=jnp.float32))
        h = jnp.maximum(h + vr[0:1], 0.0)
        h = jnp.maximum(jnp.dot(h, w2r[...], preferred_element_type=jnp.float32) + vr[1:2], 0.0)
        h = jnp.dot(h, w3r[...], preferred_element_type=jnp.float32) + vr[2:3]
        x_new = _ln(h, vr[3:4], vr[4:5]) + x_in
        x_o[...] = x_new
        xa = jnp.dot(x_new, war[...], preferred_element_type=jnp.float32)
        xb = jnp.dot(x_new, wbr[...], preferred_element_type=jnp.float32)
        xab_o[0] = _pack_bf16(xa[:, :64], xa[:, 64:])
        xab_o[1] = _pack_bf16(xb[:, :64], xb[:, 64:])

    grid = (N_NODES // _BN,)
    nspec = pl.BlockSpec((_BN, LATENT), lambda i: (i, 0))
    aspec = pl.BlockSpec((NC, _BN, LATENT), lambda i: (0, i, 0))
    return pl.pallas_call(
        body,
        grid=grid,
        in_specs=[nspec, aspec, aspec,
                  _full_spec(LATENT, LATENT), _full_spec(LATENT, LATENT),
                  _full_spec(LATENT, LATENT), _full_spec(LATENT, LATENT),
                  _full_spec(8, LATENT),
                  _full_spec(LATENT, LATENT), _full_spec(LATENT, LATENT)],
        out_specs=[nspec, pl.BlockSpec((NC, _BN, LATENT // 2), lambda i: (0, i, 0))],
        out_shape=[jax.ShapeDtypeStruct((N_NODES, LATENT), jnp.float32),
                   jax.ShapeDtypeStruct((NC, N_NODES, LATENT // 2), jnp.int32)],
        compiler_params=pltpu.CompilerParams(dimension_semantics=("parallel",)),
    )(x, agg_a, agg_b, wn1a, wn1b, wn2, wn3, vecs, wa, wb)


@jax.jit
def _tc_decoder(x, w1, w2, w3p, vecs):
    def body(x_r, w1r, w2r, w3r, vr, o):
        h = jnp.maximum(jnp.dot(x_r[...], w1r[...], preferred_element_type=jnp.float32) + vr[0:1], 0.0)
        h = jnp.maximum(jnp.dot(h, w2r[...], preferred_element_type=jnp.float32) + vr[1:2], 0.0)
        o[...] = jnp.dot(h, w3r[...], preferred_element_type=jnp.float32) + vr[2:3]

    grid = (N_NODES // _BN,)
    nspec = pl.BlockSpec((_BN, LATENT), lambda i: (i, 0))
    return pl.pallas_call(
        body,
        grid=grid,
        in_specs=[nspec,
                  _full_spec(LATENT, LATENT), _full_spec(LATENT, LATENT),
                  _full_spec(LATENT, LATENT), _full_spec(8, LATENT)],
        out_specs=nspec,
        out_shape=jax.ShapeDtypeStruct((N_NODES, LATENT), jnp.float32),
        compiler_params=pltpu.CompilerParams(dimension_semantics=("parallel",)),
    )(x, w1, w2, w3p, vecs)


def _stats_mean_std(stats):
    c = jnp.maximum(stats['count'], 1.0)
    mean = stats['sum'] / c
    std = jnp.sqrt(jnp.maximum(stats['sumsq'] / c - mean ** 2, 1e-8))
    return mean, jnp.maximum(std, 1e-8)


def _pad16(v, n):
    out = jnp.zeros((16,), jnp.float32)
    return out.at[:n].set(v).reshape(1, 16)


def _vecs(b1, b2, b3, g, bb):
    z = jnp.zeros((LATENT,), jnp.float32)
    return jnp.stack([b1, b2, b3, g, bb, z, z, z])


def kernel(world_pos, prev_world_pos, mesh_pos, params, node_type, edge_index):
    src = edge_index[0].astype(jnp.int32)
    dst = edge_index[1].astype(jnp.int32)
    npad = EPAD - N_EDGES
    src_p = jnp.concatenate([src, jnp.zeros((npad,), jnp.int32)])
    dst_pg = jnp.concatenate([dst + N_NODES,
                              jnp.full((npad,), N_NODES, jnp.int32)])
    dst_ps = jnp.concatenate([dst, jnp.full((npad,), TRASH_ROW, jnp.int32)])
    idx_g = []
    dst_s = []
    for h in range(2):
        sl = slice(h * EH, (h + 1) * EH)
        idx_g.append(jnp.concatenate([src_p[sl], dst_pg[sl]])
                     .reshape(NW, GCH, CH))
        dst_s.append(dst_ps[sl].reshape(NC, NS, SCH, CH))
    zeros128 = jnp.zeros((CH, LATENT), jnp.float32)

    ptab = jnp.zeros((N_NODES, LATENT), jnp.float32)
    ptab = ptab.at[:, 0:3].set(world_pos).at[:, 4:7].set(mesh_pos)
    pt2 = jnp.concatenate([ptab, ptab], axis=0)

    vel = world_pos - prev_world_pos
    onehot = jax.nn.one_hot(node_type, 9, dtype=jnp.float32)
    feat16 = jnp.zeros((N_NODES, 16), jnp.float32)
    feat16 = feat16.at[:, 0:3].set(vel).at[:, 3:12].set(onehot)
    nmean, nstd = _stats_mean_std(params['node_norm'])
    nmean16 = _pad16(nmean, 12)
    nsinv16 = _pad16(1.0 / nstd, 12)

    emean, estd = _stats_mean_std(params['mesh_norm'])
    emean128 = jnp.zeros((1, LATENT), jnp.float32).at[0, :8].set(emean)
    esinv128 = jnp.zeros((1, LATENT), jnp.float32).at[0, :8].set(1.0 / estd)

    def mlp_w(ps):
        return [p[0] for p in ps], [p[1] for p in ps]

    w1_0 = params['blocks'][0]['edge_mlp'][0][0]
    enc_w, enc_b = mlp_w(params['enc_node'])
    w1n16 = jnp.zeros((16, LATENT), jnp.float32).at[:12].set(enc_w[0])
    x, xab = _tc_node_encoder(
        feat16, nmean16, nsinv16, w1n16, enc_w[1], enc_w[2],
        _vecs(enc_b[0], enc_b[1], enc_b[2], *params['enc_node_ln']),
        w1_0[:LATENT], w1_0[LATENT:2 * LATENT])

    ence_w, ence_b = mlp_w(params['enc_edge'])
    w1e128 = jnp.zeros((LATENT, LATENT), jnp.float32).at[:8].set(ence_w[0])
    evecs = _vecs(ence_b[0], ence_b[1], ence_b[2], *params['enc_edge_ln'])
    e = []
    for h in range(2):
        psd = _sc_gather2(pt2, idx_g[h], jnp.float32, LATENT)
        e.append(_tc_edge_encoder(psd, emean128, esinv128, w1e128,
                                  ence_w[1], ence_w[2], evecs))

    for b in range(N_BLOCKS):
        blk = params['blocks'][b]
        ew, ebias = mlp_w(blk['edge_mlp'])
        nw_, nbias = mlp_w(blk['node_mlp'])
        ev = _vecs(ebias[0], ebias[1], ebias[2], *blk['edge_ln'])
        tab = xab.reshape(2 * N_NODES, LATENT // 2)
        xsd0 = _sc_gather2(tab, idx_g[0], jnp.int32, LATENT // 2)
        xsd1 = _sc_gather2(tab, idx_g[1], jnp.int32, LATENT // 2)
        e[0] = _tc_edge_block(xsd0, e[0], ew[0][2 * LATENT:], ew[1], ew[2], ev)
        agg0 = _sc_scatter_add(e[0], dst_s[0], zeros128)
        e[1] = _tc_edge_block(xsd1, e[1], ew[0][2 * LATENT:], ew[1], ew[2], ev)
        agg1 = _sc_scatter_add(e[1], dst_s[1], zeros128)
        nxt = params['blocks'][(b + 1) % N_BLOCKS]['edge_mlp'][0][0]
        x, xab = _tc_node_block(
            x, agg0, agg1,
            nw_[0][:LATENT], nw_[0][LATENT:], nw_[1], nw_[2],
            _vecs(nbias[0], nbias[1], nbias[2], *blk['node_ln']),
            nxt[:LATENT], nxt[LATENT:2 * LATENT])

    omean, ostd = _stats_mean_std(params['out_norm'])
    dec_w, dec_b = mlp_w(params['dec'])
    w3p = jnp.zeros((LATENT, LATENT), jnp.float32).at[:, :3].set(dec_w[2] * ostd)
    b3p = jnp.zeros((LATENT,), jnp.float32).at[:3].set(dec_b[2] * ostd + omean)
    zl = jnp.zeros((LATENT,), jnp.float32)
    pred_pad = _tc_decoder(x, dec_w[0], dec_w[1], w3p,
                           _vecs(dec_b[0], dec_b[1], b3p, zl, zl))
    acc = pred_pad[:, :3]

    pred_pos = 2.0 * world_pos + acc - prev_world_pos
    mask = (node_type == 0)[:, None]
    new_prev = jnp.where(mask, world_pos, prev_world_pos)
    new_world = jnp.where(mask, pred_pos, world_pos)
    return new_world, new_prev
